# Initial kernel scaffold; baseline (speedup 1.0000x reference)
import jax
import jax.numpy as jnp
from jax.experimental import pallas as pl


def kernel(scaffold_x, adjs, conditions, node_batch, conv0_W, conv1_W, cond0_W, cond0_b, cond1_W, cond1_b, bn1_g, bn1_b, bnc_g, bnc_b, l1_W, l1_b, bnh1_g, bnh1_b, l2_W, l2_b, bnh2_g, bnh2_b):
    raise NotImplementedError("write your pallas kernel here")


# speedup vs baseline: 1.8633x; 1.8633x over previous
import jax
import jax.numpy as jnp
from jax import lax
from jax.experimental import pallas as pl
from jax.experimental.pallas import tpu as pltpu
from jax.experimental.pallas import tpu_sc as plsc

N = 10000
D = 128
E = 320000
G = 512
CD = 16
H0 = 256
H1 = 128

NPAD = 10112
RPT = 632
CHR = 2560
WCH = 80
BLK = 8
NBLK = WCH // BLK
NBPAD = 12288


def _dg(a, w):
    return lax.dot_general(a, w, (((1,), (1,)), ((), ())),
                           preferred_element_type=jnp.float32)


def _sc_agg_body(x_hbm, srcf_hbm, dstf_hbm, zeros_hbm, cond_hbm, nb_hbm,
                 pout_hbm, cond_out_hbm,
                 idx_src, idx_dst, rows, nbidx, crows, acc, sem):
    c = lax.axis_index("c")
    s = lax.axis_index("s")
    w = c * 16 + s

    pltpu.sync_copy(zeros_hbm, acc.at[pl.ds(s * RPT, RPT)])

    @pl.when(w < 12)
    def _cond_gather():
        pltpu.sync_copy(nb_hbm.at[pl.ds(8 * w, 8)], nbidx)
        for j in range(8):
            pltpu.async_copy(cond_hbm.at[nbidx.at[j]], crows, sem).wait()
            pltpu.sync_copy(crows,
                            cond_out_hbm.at[pl.ds((8 * w + j) * 128, 128)])

    plsc.subcore_barrier()

    for r in range(4):
        def body(b, carry, r=r):
            blk = w * WCH + b * BLK
            pltpu.sync_copy(srcf_hbm.at[r, pl.ds(blk, BLK)], idx_src)
            pltpu.sync_copy(dstf_hbm.at[r, pl.ds(blk, BLK)], idx_dst)
            for j in range(BLK):
                pltpu.async_copy(x_hbm.at[idx_src.at[j]], rows, sem).wait()
                pltpu.sync_copy(rows, acc.at[idx_dst.at[j]], add=True)
            return carry

        lax.fori_loop(0, NBLK, body, 0)
        plsc.subcore_barrier()
        pltpu.sync_copy(acc.at[pl.ds(s * RPT, RPT)],
                        pout_hbm.at[c, r, pl.ds(s * RPT, RPT)])
        if r < 3:
            pltpu.sync_copy(zeros_hbm, acc.at[pl.ds(s * RPT, RPT)])
        plsc.subcore_barrier()


def _sc_agg(x, srcf, dstf, zeros, cond_pad, nbpad):
    mesh = plsc.VectorSubcoreMesh(core_axis_name="c", subcore_axis_name="s",
                                  num_cores=2, num_subcores=16)
    fn = pl.kernel(
        _sc_agg_body,
        out_type=[jax.ShapeDtypeStruct((2, 4, NPAD, D), jnp.float32),
                  jax.ShapeDtypeStruct((NBPAD, 128), jnp.float32)],
        mesh=mesh,
        scratch_types=[
            pltpu.VMEM((BLK, 128), jnp.int32),
            pltpu.VMEM((BLK, 128), jnp.int32),
            pltpu.VMEM((128, D), jnp.float32),
            pltpu.VMEM((8, 128), jnp.int32),
            pltpu.VMEM((128, 128), jnp.float32),
            pltpu.VMEM_SHARED((NPAD, D), jnp.float32),
            pltpu.SemaphoreType.DMA,
        ],
    )
    return fn(x, srcf, dstf, zeros, cond_pad, nbpad)


NB = 10
BN_ROWS = N // NB


def _acc_stats(i, t, s1_ref, s2_ref):
    @pl.when(i == 0)
    def _init():
        s1_ref[...] = jnp.zeros_like(s1_ref)
        s2_ref[...] = jnp.zeros_like(s2_ref)
    s1_ref[...] += jnp.sum(t, axis=0, keepdims=True)
    s2_ref[...] += jnp.sum(t * t, axis=0, keepdims=True)


def _norm_relu(t, s1, s2, g, b, eps):
    mu = s1 * (1.0 / N)
    var = s2 * (1.0 / N) - mu * mu
    return jnp.maximum((t - mu) * lax.rsqrt(var + eps) * g + b, 0.0)


_STAT = jax.ShapeDtypeStruct((1, D), jnp.float32)
_STAT2 = jax.ShapeDtypeStruct((1, H0), jnp.float32)
_CONST = lambda i: (0, 0)
_ROWB = lambda i: (i, 0)


def _xconv_body(x_ref, p_ref, w_ref, o_ref):
    wf = w_ref[...]
    m = _dg(x_ref[...], wf[:, 0:D])
    for r in range(4):
        m = m + _dg(p_ref[0, r] + p_ref[1, r], wf[:, D * (r + 1):D * (r + 2)])
    o_ref[...] = m


def _xconv(x, p, w):
    return pl.pallas_call(
        _xconv_body,
        grid=(NB,),
        in_specs=[pl.BlockSpec((BN_ROWS, D), _ROWB),
                  pl.BlockSpec((2, 4, BN_ROWS, D), lambda i: (0, 0, i, 0)),
                  pl.BlockSpec((D, 5 * D), _CONST)],
        out_specs=pl.BlockSpec((BN_ROWS, D), _ROWB),
        out_shape=jax.ShapeDtypeStruct((N, D), jnp.float32),
    )(x, p, w)


def _lin_body(m_ref, cnb_ref, wa_ref, wb_ref, b_ref, x_ref, s1_ref, s2_ref):
    i = pl.program_id(0)
    x = (_dg(m_ref[...], wa_ref[...]) + _dg(cnb_ref[...], wb_ref[...])
         + b_ref[...])
    x_ref[...] = x
    _acc_stats(i, x, s1_ref, s2_ref)


def _lin(m, cnb, wa, wb, b):
    return pl.pallas_call(
        _lin_body,
        grid=(NB,),
        in_specs=[pl.BlockSpec((BN_ROWS, D), _ROWB),
                  pl.BlockSpec((BN_ROWS, CD), _ROWB),
                  pl.BlockSpec((D, D), _CONST),
                  pl.BlockSpec((D, CD), _CONST),
                  pl.BlockSpec((1, D), _CONST)],
        out_specs=[pl.BlockSpec((BN_ROWS, D), _ROWB),
                   pl.BlockSpec((1, D), _CONST),
                   pl.BlockSpec((1, D), _CONST)],
        out_shape=[jax.ShapeDtypeStruct((N, D), jnp.float32), _STAT, _STAT],
    )(m, cnb, wa, wb, b)


def _c2_body(x0_ref, s1_ref, s2_ref, g_ref, b_ref, h_ref):
    h_ref[...] = _norm_relu(x0_ref[...], s1_ref[...], s2_ref[...],
                            g_ref[...], b_ref[...], 128.0)


def _c2(x0, s1, s2, g, b):
    return pl.pallas_call(
        _c2_body,
        grid=(NB,),
        in_specs=[pl.BlockSpec((BN_ROWS, D), _ROWB),
                  pl.BlockSpec((1, D), _CONST),
                  pl.BlockSpec((1, D), _CONST),
                  pl.BlockSpec((1, D), _CONST),
                  pl.BlockSpec((1, D), _CONST)],
        out_specs=pl.BlockSpec((BN_ROWS, D), _ROWB),
        out_shape=jax.ShapeDtypeStruct((N, D), jnp.float32),
    )(x0, s1, s2, g, b)


def _e2_body(x0_ref, x1_ref, s1_ref, s2_ref, t1_ref, t2_ref,
             g0_ref, b0_ref, g1_ref, b1_ref, l1w_ref, l1b_ref, cnb_ref,
             z_ref, zs1_ref, zs2_ref):
    i = pl.program_id(0)
    u0 = _norm_relu(x0_ref[...], s1_ref[...], s2_ref[...],
                    g0_ref[...], b0_ref[...], 1e-5)
    u1 = _norm_relu(x1_ref[...], t1_ref[...], t2_ref[...],
                    g1_ref[...], b1_ref[...], 1e-5)
    l1w = l1w_ref[...]
    z = (_dg(u0, l1w[:, 0:D]) + _dg(u1, l1w[:, D:2 * D])
         + _dg(cnb_ref[...], l1w[:, 2 * D:2 * D + CD]) + l1b_ref[...])
    z_ref[...] = z
    _acc_stats(i, z, zs1_ref, zs2_ref)


def _e2(x0, x1, s1, s2, t1, t2, g0, b0, g1, b1, l1w, l1b, cnb):
    return pl.pallas_call(
        _e2_body,
        grid=(NB,),
        in_specs=[pl.BlockSpec((BN_ROWS, D), _ROWB),
                  pl.BlockSpec((BN_ROWS, D), _ROWB),
                  pl.BlockSpec((1, D), _CONST),
                  pl.BlockSpec((1, D), _CONST),
                  pl.BlockSpec((1, D), _CONST),
                  pl.BlockSpec((1, D), _CONST),
                  pl.BlockSpec((1, D), _CONST),
                  pl.BlockSpec((1, D), _CONST),
                  pl.BlockSpec((1, D), _CONST),
                  pl.BlockSpec((1, D), _CONST),
                  pl.BlockSpec((H0, 2 * D + CD), _CONST),
                  pl.BlockSpec((1, H0), _CONST),
                  pl.BlockSpec((BN_ROWS, CD), _ROWB)],
        out_specs=[pl.BlockSpec((BN_ROWS, H0), _ROWB),
                   pl.BlockSpec((1, H0), _CONST),
                   pl.BlockSpec((1, H0), _CONST)],
        out_shape=[jax.ShapeDtypeStruct((N, H0), jnp.float32), _STAT2, _STAT2],
    )(x0, x1, s1, s2, t1, t2, g0, b0, g1, b1, l1w, l1b, cnb)


def _e3_body(z_ref, zs1_ref, zs2_ref, g_ref, b_ref, l2w_ref, l2b_ref,
             o_ref, os1_ref, os2_ref):
    i = pl.program_id(0)
    zz = _norm_relu(z_ref[...], zs1_ref[...], zs2_ref[...],
                    g_ref[...], b_ref[...], 1e-5)
    o = _dg(zz, l2w_ref[...]) + l2b_ref[...]
    o_ref[...] = o
    _acc_stats(i, o, os1_ref, os2_ref)


def _e3(z, zs1, zs2, g, b, l2w, l2b):
    return pl.pallas_call(
        _e3_body,
        grid=(NB,),
        in_specs=[pl.BlockSpec((BN_ROWS, H0), _ROWB),
                  pl.BlockSpec((1, H0), _CONST),
                  pl.BlockSpec((1, H0), _CONST),
                  pl.BlockSpec((1, H0), _CONST),
                  pl.BlockSpec((1, H0), _CONST),
                  pl.BlockSpec((H1, H0), _CONST),
                  pl.BlockSpec((1, H1), _CONST)],
        out_specs=[pl.BlockSpec((BN_ROWS, H1), _ROWB),
                   pl.BlockSpec((1, H1), _CONST),
                   pl.BlockSpec((1, H1), _CONST)],
        out_shape=[jax.ShapeDtypeStruct((N, H1), jnp.float32), _STAT, _STAT],
    )(z, zs1, zs2, g, b, l2w, l2b)


def _e4_body(o_ref, s1_ref, s2_ref, g_ref, b_ref, out_ref):
    out_ref[...] = _norm_relu(o_ref[...], s1_ref[...], s2_ref[...],
                              g_ref[...], b_ref[...], 1e-5)


def _e4(o, s1, s2, g, b):
    return pl.pallas_call(
        _e4_body,
        grid=(NB,),
        in_specs=[pl.BlockSpec((BN_ROWS, H1), _ROWB),
                  pl.BlockSpec((1, H1), _CONST),
                  pl.BlockSpec((1, H1), _CONST),
                  pl.BlockSpec((1, H1), _CONST),
                  pl.BlockSpec((1, H1), _CONST)],
        out_specs=pl.BlockSpec((BN_ROWS, H1), _ROWB),
        out_shape=jax.ShapeDtypeStruct((N, H1), jnp.float32),
    )(o, s1, s2, g, b)


def kernel(scaffold_x, adjs, conditions, node_batch, conv0_W, conv1_W,
           cond0_W, cond0_b, cond1_W, cond1_b, bn1_g, bn1_b, bnc_g, bnc_b,
           l1_W, l1_b, bnh1_g, bnh1_b, l2_W, l2_b, bnh2_g, bnh2_b):
    adjs = adjs.astype(jnp.int32)
    src4 = adjs[:, 0, :].reshape(4, CHR - 60, 128)
    dst4 = adjs[:, 1, :].reshape(4, CHR - 60, 128)
    srcf = jnp.pad(src4, ((0, 0), (0, 60), (0, 0)))
    dstf = jnp.pad(dst4, ((0, 0), (0, 60), (0, 0)), constant_values=N)
    nbpad = (jnp.zeros((NBPAD,), jnp.int32)
             .at[:N].set(node_batch.astype(jnp.int32)).reshape(NBPAD // 128, 128))
    zeros = jnp.zeros((RPT, D), jnp.float32)
    cond_pad = jnp.zeros((G, 128), jnp.float32).at[:, :CD].set(conditions)

    p, cond_out = _sc_agg(scaffold_x, srcf, dstf, zeros, cond_pad, nbpad)
    cnb = cond_out[:N, :CD]
    x0m = _xconv(scaffold_x, p, conv0_W)
    x0, s1, s2 = _lin(x0m, cnb, cond0_W[:, :D], cond0_W[:, D:],
                      cond0_b.reshape(1, D))
    h = _c2(x0, s1, s2, bn1_g.reshape(1, D), bn1_b.reshape(1, D))
    q, _ = _sc_agg(h, srcf, dstf, zeros, cond_pad, nbpad)
    x1m = _xconv(h, q, conv1_W)
    x1, t1, t2 = _lin(x1m, cnb, cond1_W[:, :D], cond1_W[:, D:],
                      cond1_b.reshape(1, D))
    z, zs1, zs2 = _e2(x0, x1, s1, s2, t1, t2,
                      bnc_g[:D].reshape(1, D), bnc_b[:D].reshape(1, D),
                      bnc_g[D:].reshape(1, D), bnc_b[D:].reshape(1, D),
                      l1_W, l1_b.reshape(1, H0), cnb)
    o, os1, os2 = _e3(z, zs1, zs2, bnh1_g.reshape(1, H0),
                      bnh1_b.reshape(1, H0), l2_W, l2_b.reshape(1, H1))
    out = _e4(o, os1, os2, bnh2_g.reshape(1, H1), bnh2_b.reshape(1, H1))
    return out


# speedup vs baseline: 4.6258x; 2.4826x over previous
import functools

import jax
import jax.numpy as jnp
from jax import lax
from jax.experimental import pallas as pl
from jax.experimental.pallas import tpu as pltpu
from jax.experimental.pallas import tpu_sc as plsc

N = 10000
D = 128
E = 320000
G = 512
CD = 16
H0 = 256
H1 = 128

NPAD = 10112
RPT = 632
CHR = 2560
WCH = 80
BLK = 8
NBLK = WCH // BLK
NBPAD = 12288


def _dg(a, w):
    return lax.dot_general(a, w, (((1,), (1,)), ((), ())),
                           preferred_element_type=jnp.float32)


def _sc_agg_body(x_hbm, srcf_hbm, dstf_hbm, zeros_hbm, cond_hbm, nb_hbm,
                 pout_hbm, cond_out_hbm,
                 idx_src, idx_dst, rows, nbidx, crows, acc, sem,
                 with_cond=True):
    c = lax.axis_index("c")
    s = lax.axis_index("s")
    w = c * 16 + s

    pltpu.sync_copy(zeros_hbm, acc.at[pl.ds(s * RPT, RPT)])

    if with_cond:
        @pl.when(s >= 10)
        def _cond_gather():
            cw = c * 6 + (s - 10)
            pltpu.sync_copy(nb_hbm.at[pl.ds(8 * cw, 8)], nbidx)
            for j in range(8):
                pltpu.async_copy(cond_hbm.at[nbidx.at[j]], crows, sem).wait()
                pltpu.sync_copy(
                    crows, cond_out_hbm.at[pl.ds((8 * cw + j) * 128, 128)])

    plsc.subcore_barrier()

    for r in range(4):
        def body(b, carry, r=r):
            blk = w * WCH + b * BLK
            pltpu.sync_copy(srcf_hbm.at[r, pl.ds(blk, BLK)], idx_src)
            pltpu.sync_copy(dstf_hbm.at[r, pl.ds(blk, BLK)], idx_dst)
            for j in range(BLK):
                pltpu.async_copy(x_hbm.at[idx_src.at[j]], rows, sem).wait()
                pltpu.sync_copy(rows, acc.at[idx_dst.at[j]], add=True)
            return carry

        lax.fori_loop(0, NBLK, body, 0)
        plsc.subcore_barrier()
        pltpu.sync_copy(acc.at[pl.ds(s * RPT, RPT)],
                        pout_hbm.at[c, r, pl.ds(s * RPT, RPT)])
        if r < 3:
            pltpu.sync_copy(zeros_hbm, acc.at[pl.ds(s * RPT, RPT)])
        plsc.subcore_barrier()


def _sc_agg(x, srcf, dstf, zeros, cond_pad, nbpad, with_cond=True):
    mesh = plsc.VectorSubcoreMesh(core_axis_name="c", subcore_axis_name="s",
                                  num_cores=2, num_subcores=16)
    fn = pl.kernel(
        functools.partial(_sc_agg_body, with_cond=with_cond),
        out_type=[jax.ShapeDtypeStruct((2, 4, NPAD, D), jnp.float32),
                  jax.ShapeDtypeStruct((NBPAD, 128), jnp.float32)],
        mesh=mesh,
        scratch_types=[
            pltpu.VMEM((BLK, 128), jnp.int32),
            pltpu.VMEM((BLK, 128), jnp.int32),
            pltpu.VMEM((128, D), jnp.float32),
            pltpu.VMEM((8, 128), jnp.int32),
            pltpu.VMEM((128, 128), jnp.float32),
            pltpu.VMEM_SHARED((NPAD, D), jnp.float32),
            pltpu.SemaphoreType.DMA,
        ],
    )
    return fn(x, srcf, dstf, zeros, cond_pad, nbpad)


NB = 10
BN_ROWS = N // NB


def _acc_stats(i, t, s1_ref, s2_ref):
    @pl.when(i == 0)
    def _init():
        s1_ref[...] = jnp.zeros_like(s1_ref)
        s2_ref[...] = jnp.zeros_like(s2_ref)
    s1_ref[...] += jnp.sum(t, axis=0, keepdims=True)
    s2_ref[...] += jnp.sum(t * t, axis=0, keepdims=True)


def _norm_relu(t, s1, s2, g, b, eps):
    mu = s1 * (1.0 / N)
    var = s2 * (1.0 / N) - mu * mu
    return jnp.maximum((t - mu) * lax.rsqrt(var + eps) * g + b, 0.0)


_STAT = jax.ShapeDtypeStruct((1, D), jnp.float32)
_STAT2 = jax.ShapeDtypeStruct((1, H0), jnp.float32)
_CONST = lambda i: (0, 0)
_ROWB = lambda i: (i, 0)


def _xconv_body(x_ref, p_ref, w_ref, o_ref):
    wf = w_ref[...]
    m = _dg(x_ref[...], wf[:, 0:D])
    for r in range(4):
        m = m + _dg(p_ref[0, r] + p_ref[1, r], wf[:, D * (r + 1):D * (r + 2)])
    o_ref[...] = m


def _xconv(x, p, w):
    return pl.pallas_call(
        _xconv_body,
        grid=(NB,),
        in_specs=[pl.BlockSpec((BN_ROWS, D), _ROWB),
                  pl.BlockSpec((2, 4, BN_ROWS, D), lambda i: (0, 0, i, 0)),
                  pl.BlockSpec((D, 5 * D), _CONST)],
        out_specs=pl.BlockSpec((BN_ROWS, D), _ROWB),
        out_shape=jax.ShapeDtypeStruct((N, D), jnp.float32),
    )(x, p, w)


def _lin_body(m_ref, cnb_ref, wa_ref, wb_ref, b_ref, x_ref, s1_ref, s2_ref):
    i = pl.program_id(0)
    x = (_dg(m_ref[...], wa_ref[...]) + _dg(cnb_ref[...], wb_ref[...])
         + b_ref[...])
    x_ref[...] = x
    _acc_stats(i, x, s1_ref, s2_ref)


def _lin(m, cnb, wa, wb, b):
    return pl.pallas_call(
        _lin_body,
        grid=(NB,),
        in_specs=[pl.BlockSpec((BN_ROWS, D), _ROWB),
                  pl.BlockSpec((BN_ROWS, CD), _ROWB),
                  pl.BlockSpec((D, D), _CONST),
                  pl.BlockSpec((D, CD), _CONST),
                  pl.BlockSpec((1, D), _CONST)],
        out_specs=[pl.BlockSpec((BN_ROWS, D), _ROWB),
                   pl.BlockSpec((1, D), _CONST),
                   pl.BlockSpec((1, D), _CONST)],
        out_shape=[jax.ShapeDtypeStruct((N, D), jnp.float32), _STAT, _STAT],
    )(m, cnb, wa, wb, b)


def _c2_body(x0_ref, s1_ref, s2_ref, g_ref, b_ref, h_ref):
    h_ref[...] = _norm_relu(x0_ref[...], s1_ref[...], s2_ref[...],
                            g_ref[...], b_ref[...], 128.0)


def _c2(x0, s1, s2, g, b):
    return pl.pallas_call(
        _c2_body,
        grid=(NB,),
        in_specs=[pl.BlockSpec((BN_ROWS, D), _ROWB),
                  pl.BlockSpec((1, D), _CONST),
                  pl.BlockSpec((1, D), _CONST),
                  pl.BlockSpec((1, D), _CONST),
                  pl.BlockSpec((1, D), _CONST)],
        out_specs=pl.BlockSpec((BN_ROWS, D), _ROWB),
        out_shape=jax.ShapeDtypeStruct((N, D), jnp.float32),
    )(x0, s1, s2, g, b)


def _e2_body(x0_ref, x1_ref, s1_ref, s2_ref, t1_ref, t2_ref,
             g0_ref, b0_ref, g1_ref, b1_ref, l1w_ref, l1b_ref, cnb_ref,
             z_ref, zs1_ref, zs2_ref):
    i = pl.program_id(0)
    u0 = _norm_relu(x0_ref[...], s1_ref[...], s2_ref[...],
                    g0_ref[...], b0_ref[...], 1e-5)
    u1 = _norm_relu(x1_ref[...], t1_ref[...], t2_ref[...],
                    g1_ref[...], b1_ref[...], 1e-5)
    l1w = l1w_ref[...]
    z = (_dg(u0, l1w[:, 0:D]) + _dg(u1, l1w[:, D:2 * D])
         + _dg(cnb_ref[...], l1w[:, 2 * D:2 * D + CD]) + l1b_ref[...])
    z_ref[...] = z
    _acc_stats(i, z, zs1_ref, zs2_ref)


def _e2(x0, x1, s1, s2, t1, t2, g0, b0, g1, b1, l1w, l1b, cnb):
    return pl.pallas_call(
        _e2_body,
        grid=(NB,),
        in_specs=[pl.BlockSpec((BN_ROWS, D), _ROWB),
                  pl.BlockSpec((BN_ROWS, D), _ROWB),
                  pl.BlockSpec((1, D), _CONST),
                  pl.BlockSpec((1, D), _CONST),
                  pl.BlockSpec((1, D), _CONST),
                  pl.BlockSpec((1, D), _CONST),
                  pl.BlockSpec((1, D), _CONST),
                  pl.BlockSpec((1, D), _CONST),
                  pl.BlockSpec((1, D), _CONST),
                  pl.BlockSpec((1, D), _CONST),
                  pl.BlockSpec((H0, 2 * D + CD), _CONST),
                  pl.BlockSpec((1, H0), _CONST),
                  pl.BlockSpec((BN_ROWS, CD), _ROWB)],
        out_specs=[pl.BlockSpec((BN_ROWS, H0), _ROWB),
                   pl.BlockSpec((1, H0), _CONST),
                   pl.BlockSpec((1, H0), _CONST)],
        out_shape=[jax.ShapeDtypeStruct((N, H0), jnp.float32), _STAT2, _STAT2],
    )(x0, x1, s1, s2, t1, t2, g0, b0, g1, b1, l1w, l1b, cnb)


def _e3_body(z_ref, zs1_ref, zs2_ref, g_ref, b_ref, l2w_ref, l2b_ref,
             o_ref, os1_ref, os2_ref):
    i = pl.program_id(0)
    zz = _norm_relu(z_ref[...], zs1_ref[...], zs2_ref[...],
                    g_ref[...], b_ref[...], 1e-5)
    o = _dg(zz, l2w_ref[...]) + l2b_ref[...]
    o_ref[...] = o
    _acc_stats(i, o, os1_ref, os2_ref)


def _e3(z, zs1, zs2, g, b, l2w, l2b):
    return pl.pallas_call(
        _e3_body,
        grid=(NB,),
        in_specs=[pl.BlockSpec((BN_ROWS, H0), _ROWB),
                  pl.BlockSpec((1, H0), _CONST),
                  pl.BlockSpec((1, H0), _CONST),
                  pl.BlockSpec((1, H0), _CONST),
                  pl.BlockSpec((1, H0), _CONST),
                  pl.BlockSpec((H1, H0), _CONST),
                  pl.BlockSpec((1, H1), _CONST)],
        out_specs=[pl.BlockSpec((BN_ROWS, H1), _ROWB),
                   pl.BlockSpec((1, H1), _CONST),
                   pl.BlockSpec((1, H1), _CONST)],
        out_shape=[jax.ShapeDtypeStruct((N, H1), jnp.float32), _STAT, _STAT],
    )(z, zs1, zs2, g, b, l2w, l2b)


def _e4_body(o_ref, s1_ref, s2_ref, g_ref, b_ref, out_ref):
    out_ref[...] = _norm_relu(o_ref[...], s1_ref[...], s2_ref[...],
                              g_ref[...], b_ref[...], 1e-5)


def _e4(o, s1, s2, g, b):
    return pl.pallas_call(
        _e4_body,
        grid=(NB,),
        in_specs=[pl.BlockSpec((BN_ROWS, H1), _ROWB),
                  pl.BlockSpec((1, H1), _CONST),
                  pl.BlockSpec((1, H1), _CONST),
                  pl.BlockSpec((1, H1), _CONST),
                  pl.BlockSpec((1, H1), _CONST)],
        out_specs=pl.BlockSpec((BN_ROWS, H1), _ROWB),
        out_shape=jax.ShapeDtypeStruct((N, H1), jnp.float32),
    )(o, s1, s2, g, b)


def kernel(scaffold_x, adjs, conditions, node_batch, conv0_W, conv1_W,
           cond0_W, cond0_b, cond1_W, cond1_b, bn1_g, bn1_b, bnc_g, bnc_b,
           l1_W, l1_b, bnh1_g, bnh1_b, l2_W, l2_b, bnh2_g, bnh2_b):
    adjs = adjs.astype(jnp.int32)
    src4 = adjs[:, 0, :].reshape(4, CHR - 60, 128)
    dst4 = adjs[:, 1, :].reshape(4, CHR - 60, 128)
    lane = jnp.arange(128, dtype=jnp.int32)
    padsrc = jnp.broadcast_to(lane, (4, 60, 128))
    paddst = jnp.broadcast_to(N + lane % (NPAD - N), (4, 60, 128))
    srcf = jnp.concatenate([src4, padsrc], axis=1)
    dstf = jnp.concatenate([dst4, paddst], axis=1)
    nbpad = (jnp.zeros((NBPAD,), jnp.int32)
             .at[:N].set(node_batch.astype(jnp.int32)).reshape(NBPAD // 128, 128))
    zeros = jnp.zeros((RPT, D), jnp.float32)
    cond_pad = jnp.zeros((G, 128), jnp.float32).at[:, :CD].set(conditions)

    p, cond_out = _sc_agg(scaffold_x, srcf, dstf, zeros, cond_pad, nbpad)
    cnb = cond_out[:N, :CD]
    x0m = _xconv(scaffold_x, p, conv0_W)
    x0, s1, s2 = _lin(x0m, cnb, cond0_W[:, :D], cond0_W[:, D:],
                      cond0_b.reshape(1, D))
    h = _c2(x0, s1, s2, bn1_g.reshape(1, D), bn1_b.reshape(1, D))
    q, _ = _sc_agg(h, srcf, dstf, zeros, cond_pad, nbpad, with_cond=False)
    x1m = _xconv(h, q, conv1_W)
    x1, t1, t2 = _lin(x1m, cnb, cond1_W[:, :D], cond1_W[:, D:],
                      cond1_b.reshape(1, D))
    z, zs1, zs2 = _e2(x0, x1, s1, s2, t1, t2,
                      bnc_g[:D].reshape(1, D), bnc_b[:D].reshape(1, D),
                      bnc_g[D:].reshape(1, D), bnc_b[D:].reshape(1, D),
                      l1_W, l1_b.reshape(1, H0), cnb)
    o, os1, os2 = _e3(z, zs1, zs2, bnh1_g.reshape(1, H0),
                      bnh1_b.reshape(1, H0), l2_W, l2_b.reshape(1, H1))
    out = _e4(o, os1, os2, bnh2_g.reshape(1, H1), bnh2_b.reshape(1, H1))
    return out


# speedup vs baseline: 6.1267x; 1.3245x over previous
import functools

import jax
import jax.numpy as jnp
from jax import lax
from jax.experimental import pallas as pl
from jax.experimental.pallas import tpu as pltpu
from jax.experimental.pallas import tpu_sc as plsc

N = 10000
D = 128
E = 320000
G = 512
CD = 16
H0 = 256
H1 = 128

NPAD = 10112
RPT = 632
CHR = 2560
WCH = 80
BLK = 8
NBLK = WCH // BLK
NBPAD = 12288


def _dg(a, w):
    return lax.dot_general(a, w, (((1,), (1,)), ((), ())),
                           preferred_element_type=jnp.float32)


def _sc_agg_body(x_hbm, srcf_hbm, dstf_hbm, zeros_hbm, pout_hbm,
                 idx_src, idx_dst, rows, acc, gsem, ssem):
    c = lax.axis_index("c")
    s = lax.axis_index("s")
    w = c * 16 + s

    pltpu.sync_copy(zeros_hbm, acc.at[pl.ds(s * RPT, RPT)])
    plsc.subcore_barrier()

    for r in range(4):
        def body(b, carry, r=r):
            blk = w * WCH + b * BLK
            pltpu.sync_copy(srcf_hbm.at[r, pl.ds(blk, BLK)], idx_src)
            pltpu.sync_copy(dstf_hbm.at[r, pl.ds(blk, BLK)], idx_dst)
            pltpu.async_copy(x_hbm.at[idx_src.at[0]], rows.at[0], gsem)
            for j in range(BLK):
                k = j % 2
                pltpu.make_async_copy(x_hbm.at[idx_src.at[j]],
                                      rows.at[k], gsem).wait()
                if j < BLK - 1:
                    if j >= 1:
                        pltpu.make_async_copy(
                            rows.at[1 - k],
                            acc.at[idx_dst.at[j - 1]], ssem).wait()
                    pltpu.async_copy(x_hbm.at[idx_src.at[j + 1]],
                                     rows.at[1 - k], gsem)
                pltpu.async_copy(rows.at[k], acc.at[idx_dst.at[j]], ssem,
                                 add=True)
            for j in (BLK - 2, BLK - 1):
                pltpu.make_async_copy(rows.at[j % 2],
                                      acc.at[idx_dst.at[j]], ssem).wait()
            return carry

        lax.fori_loop(0, NBLK, body, 0)
        plsc.subcore_barrier()
        pltpu.sync_copy(acc.at[pl.ds(s * RPT, RPT)],
                        pout_hbm.at[c, r, pl.ds(s * RPT, RPT)])
        if r < 3:
            pltpu.sync_copy(zeros_hbm, acc.at[pl.ds(s * RPT, RPT)])
        plsc.subcore_barrier()


def _sc_agg(x, srcf, dstf, zeros):
    mesh = plsc.VectorSubcoreMesh(core_axis_name="c", subcore_axis_name="s",
                                  num_cores=2, num_subcores=16)
    fn = pl.kernel(
        _sc_agg_body,
        out_type=jax.ShapeDtypeStruct((2, 4, NPAD, D), jnp.float32),
        mesh=mesh,
        scratch_types=[
            pltpu.VMEM((BLK, 128), jnp.int32),
            pltpu.VMEM((BLK, 128), jnp.int32),
            pltpu.VMEM((2, 128, D), jnp.float32),
            pltpu.VMEM_SHARED((NPAD, D), jnp.float32),
            pltpu.SemaphoreType.DMA,
            pltpu.SemaphoreType.DMA,
        ],
    )
    return fn(x, srcf, dstf, zeros)


NB = 10
BN_ROWS = N // NB


def _acc_stats(i, t, s1_ref, s2_ref):
    @pl.when(i == 0)
    def _init():
        s1_ref[...] = jnp.zeros_like(s1_ref)
        s2_ref[...] = jnp.zeros_like(s2_ref)
    s1_ref[...] += jnp.sum(t, axis=0, keepdims=True)
    s2_ref[...] += jnp.sum(t * t, axis=0, keepdims=True)


def _norm_relu(t, s1, s2, g, b, eps):
    mu = s1 * (1.0 / N)
    var = s2 * (1.0 / N) - mu * mu
    return jnp.maximum((t - mu) * lax.rsqrt(var + eps) * g + b, 0.0)


_STAT = jax.ShapeDtypeStruct((1, D), jnp.float32)
_STAT2 = jax.ShapeDtypeStruct((1, H0), jnp.float32)
_CONST = lambda i: (0, 0)
_ROWB = lambda i: (i, 0)


def _xconv_body(x_ref, p_ref, w_ref, o_ref):
    wf = w_ref[...]
    m = _dg(x_ref[...], wf[:, 0:D])
    for r in range(4):
        m = m + _dg(p_ref[0, r] + p_ref[1, r], wf[:, D * (r + 1):D * (r + 2)])
    o_ref[...] = m


def _xconv(x, p, w):
    return pl.pallas_call(
        _xconv_body,
        grid=(NB,),
        in_specs=[pl.BlockSpec((BN_ROWS, D), _ROWB),
                  pl.BlockSpec((2, 4, BN_ROWS, D), lambda i: (0, 0, i, 0)),
                  pl.BlockSpec((D, 5 * D), _CONST)],
        out_specs=pl.BlockSpec((BN_ROWS, D), _ROWB),
        out_shape=jax.ShapeDtypeStruct((N, D), jnp.float32),
    )(x, p, w)


def _lin_body(m_ref, nb_ref, cond_ref, wa_ref, wb_ref, b_ref,
              x_ref, cnb_ref, s1_ref, s2_ref):
    i = pl.program_id(0)
    nb = nb_ref[...][:, 0:1]
    iota = lax.broadcasted_iota(jnp.int32, (BN_ROWS, G), 1)
    onehot = (nb == iota).astype(jnp.float32)
    cnb = lax.dot_general(onehot, cond_ref[...], (((1,), (0,)), ((), ())),
                          preferred_element_type=jnp.float32)
    cnb_ref[...] = cnb
    x = _dg(m_ref[...], wa_ref[...]) + _dg(cnb, wb_ref[...]) + b_ref[...]
    x_ref[...] = x
    _acc_stats(i, x, s1_ref, s2_ref)


def _lin(m, nb, cond, wa, wb, b):
    return pl.pallas_call(
        _lin_body,
        grid=(NB,),
        in_specs=[pl.BlockSpec((BN_ROWS, D), _ROWB),
                  pl.BlockSpec((BN_ROWS, 1), _ROWB),
                  pl.BlockSpec((G, CD), _CONST),
                  pl.BlockSpec((D, D), _CONST),
                  pl.BlockSpec((D, CD), _CONST),
                  pl.BlockSpec((1, D), _CONST)],
        out_specs=[pl.BlockSpec((BN_ROWS, D), _ROWB),
                   pl.BlockSpec((BN_ROWS, CD), _ROWB),
                   pl.BlockSpec((1, D), _CONST),
                   pl.BlockSpec((1, D), _CONST)],
        out_shape=[jax.ShapeDtypeStruct((N, D), jnp.float32),
                   jax.ShapeDtypeStruct((N, CD), jnp.float32), _STAT, _STAT],
    )(m, nb, cond, wa, wb, b)


def _c2_body(x0_ref, s1_ref, s2_ref, g_ref, b_ref, h_ref):
    h_ref[...] = _norm_relu(x0_ref[...], s1_ref[...], s2_ref[...],
                            g_ref[...], b_ref[...], 128.0)


def _c2(x0, s1, s2, g, b):
    return pl.pallas_call(
        _c2_body,
        grid=(NB,),
        in_specs=[pl.BlockSpec((BN_ROWS, D), _ROWB),
                  pl.BlockSpec((1, D), _CONST),
                  pl.BlockSpec((1, D), _CONST),
                  pl.BlockSpec((1, D), _CONST),
                  pl.BlockSpec((1, D), _CONST)],
        out_specs=pl.BlockSpec((BN_ROWS, D), _ROWB),
        out_shape=jax.ShapeDtypeStruct((N, D), jnp.float32),
    )(x0, s1, s2, g, b)


def _e2_body(x0_ref, x1_ref, s1_ref, s2_ref, t1_ref, t2_ref,
             g0_ref, b0_ref, g1_ref, b1_ref, l1w_ref, l1b_ref, cnb_ref,
             z_ref, zs1_ref, zs2_ref):
    i = pl.program_id(0)
    u0 = _norm_relu(x0_ref[...], s1_ref[...], s2_ref[...],
                    g0_ref[...], b0_ref[...], 1e-5)
    u1 = _norm_relu(x1_ref[...], t1_ref[...], t2_ref[...],
                    g1_ref[...], b1_ref[...], 1e-5)
    l1w = l1w_ref[...]
    z = (_dg(u0, l1w[:, 0:D]) + _dg(u1, l1w[:, D:2 * D])
         + _dg(cnb_ref[...], l1w[:, 2 * D:2 * D + CD]) + l1b_ref[...])
    z_ref[...] = z
    _acc_stats(i, z, zs1_ref, zs2_ref)


def _e2(x0, x1, s1, s2, t1, t2, g0, b0, g1, b1, l1w, l1b, cnb):
    return pl.pallas_call(
        _e2_body,
        grid=(NB,),
        in_specs=[pl.BlockSpec((BN_ROWS, D), _ROWB),
                  pl.BlockSpec((BN_ROWS, D), _ROWB),
                  pl.BlockSpec((1, D), _CONST),
                  pl.BlockSpec((1, D), _CONST),
                  pl.BlockSpec((1, D), _CONST),
                  pl.BlockSpec((1, D), _CONST),
                  pl.BlockSpec((1, D), _CONST),
                  pl.BlockSpec((1, D), _CONST),
                  pl.BlockSpec((1, D), _CONST),
                  pl.BlockSpec((1, D), _CONST),
                  pl.BlockSpec((H0, 2 * D + CD), _CONST),
                  pl.BlockSpec((1, H0), _CONST),
                  pl.BlockSpec((BN_ROWS, CD), _ROWB)],
        out_specs=[pl.BlockSpec((BN_ROWS, H0), _ROWB),
                   pl.BlockSpec((1, H0), _CONST),
                   pl.BlockSpec((1, H0), _CONST)],
        out_shape=[jax.ShapeDtypeStruct((N, H0), jnp.float32), _STAT2, _STAT2],
    )(x0, x1, s1, s2, t1, t2, g0, b0, g1, b1, l1w, l1b, cnb)


def _e3_body(z_ref, zs1_ref, zs2_ref, g_ref, b_ref, l2w_ref, l2b_ref,
             o_ref, os1_ref, os2_ref):
    i = pl.program_id(0)
    zz = _norm_relu(z_ref[...], zs1_ref[...], zs2_ref[...],
                    g_ref[...], b_ref[...], 1e-5)
    o = _dg(zz, l2w_ref[...]) + l2b_ref[...]
    o_ref[...] = o
    _acc_stats(i, o, os1_ref, os2_ref)


def _e3(z, zs1, zs2, g, b, l2w, l2b):
    return pl.pallas_call(
        _e3_body,
        grid=(NB,),
        in_specs=[pl.BlockSpec((BN_ROWS, H0), _ROWB),
                  pl.BlockSpec((1, H0), _CONST),
                  pl.BlockSpec((1, H0), _CONST),
                  pl.BlockSpec((1, H0), _CONST),
                  pl.BlockSpec((1, H0), _CONST),
                  pl.BlockSpec((H1, H0), _CONST),
                  pl.BlockSpec((1, H1), _CONST)],
        out_specs=[pl.BlockSpec((BN_ROWS, H1), _ROWB),
                   pl.BlockSpec((1, H1), _CONST),
                   pl.BlockSpec((1, H1), _CONST)],
        out_shape=[jax.ShapeDtypeStruct((N, H1), jnp.float32), _STAT, _STAT],
    )(z, zs1, zs2, g, b, l2w, l2b)


def _e4_body(o_ref, s1_ref, s2_ref, g_ref, b_ref, out_ref):
    out_ref[...] = _norm_relu(o_ref[...], s1_ref[...], s2_ref[...],
                              g_ref[...], b_ref[...], 1e-5)


def _e4(o, s1, s2, g, b):
    return pl.pallas_call(
        _e4_body,
        grid=(NB,),
        in_specs=[pl.BlockSpec((BN_ROWS, H1), _ROWB),
                  pl.BlockSpec((1, H1), _CONST),
                  pl.BlockSpec((1, H1), _CONST),
                  pl.BlockSpec((1, H1), _CONST),
                  pl.BlockSpec((1, H1), _CONST)],
        out_specs=pl.BlockSpec((BN_ROWS, H1), _ROWB),
        out_shape=jax.ShapeDtypeStruct((N, H1), jnp.float32),
    )(o, s1, s2, g, b)


def kernel(scaffold_x, adjs, conditions, node_batch, conv0_W, conv1_W,
           cond0_W, cond0_b, cond1_W, cond1_b, bn1_g, bn1_b, bnc_g, bnc_b,
           l1_W, l1_b, bnh1_g, bnh1_b, l2_W, l2_b, bnh2_g, bnh2_b):
    adjs = adjs.astype(jnp.int32)
    src4 = adjs[:, 0, :].reshape(4, CHR - 60, 128)
    dst4 = adjs[:, 1, :].reshape(4, CHR - 60, 128)
    lane = jnp.arange(128, dtype=jnp.int32)
    padsrc = jnp.broadcast_to(lane, (4, 60, 128))
    paddst = jnp.broadcast_to(N + lane % (NPAD - N), (4, 60, 128))
    srcf = jnp.concatenate([src4, padsrc], axis=1)
    dstf = jnp.concatenate([dst4, paddst], axis=1)
    zeros = jnp.zeros((RPT, D), jnp.float32)
    nb2d = node_batch.astype(jnp.int32).reshape(N, 1)

    p = _sc_agg(scaffold_x, srcf, dstf, zeros)
    x0m = _xconv(scaffold_x, p, conv0_W)
    x0, cnb, s1, s2 = _lin(x0m, nb2d, conditions, cond0_W[:, :D],
                           cond0_W[:, D:], cond0_b.reshape(1, D))
    h = _c2(x0, s1, s2, bn1_g.reshape(1, D), bn1_b.reshape(1, D))
    q = _sc_agg(h, srcf, dstf, zeros)
    x1m = _xconv(h, q, conv1_W)
    x1, _, t1, t2 = _lin(x1m, nb2d, conditions, cond1_W[:, :D],
                         cond1_W[:, D:], cond1_b.reshape(1, D))
    z, zs1, zs2 = _e2(x0, x1, s1, s2, t1, t2,
                      bnc_g[:D].reshape(1, D), bnc_b[:D].reshape(1, D),
                      bnc_g[D:].reshape(1, D), bnc_b[D:].reshape(1, D),
                      l1_W, l1_b.reshape(1, H0), cnb)
    o, os1, os2 = _e3(z, zs1, zs2, bnh1_g.reshape(1, H0),
                      bnh1_b.reshape(1, H0), l2_W, l2_b.reshape(1, H1))
    out = _e4(o, os1, os2, bnh2_g.reshape(1, H1), bnh2_b.reshape(1, H1))
    return out
